# baseline (speedup 1.0000x reference)
import functools

import jax
import jax.numpy as jnp
from jax import lax
from jax.experimental import pallas as pl
from jax.experimental.pallas import tpu as pltpu
from jax.experimental.pallas import tpu_sc as plsc

N = 10000
E = 160000
D = 256
H = 256
G = 128
OUT = 10

NC = 2
NS = 16
NW = NC * NS
HH = H // 2

EPT = E // NW
EPS = E // NS
CH = 40
NF = 252
NB = 6
PADE = NF * CH - EPS
RPT = N // NS
BM = 1000


def _make_deg_kernel():
    mesh = plsc.VectorSubcoreMesh(core_axis_name="c", subcore_axis_name="s")

    @functools.partial(
        pl.kernel,
        out_type=jax.ShapeDtypeStruct((NW, N), jnp.float32),
        mesh=mesh,
        scratch_types=[
            pltpu.VMEM((EPT + 16,), jnp.int32),
            pltpu.VMEM((N,), jnp.float32),
        ],
        compiler_params=pltpu.CompilerParams(needs_layout_passes=False,
                                             use_tc_tiling_on_sc=False),
    )
    def deg_kernel(dst_hbm, out_hbm, dst_v, deg_v):
        c = lax.axis_index("c")
        s = lax.axis_index("s")
        wid = s * NC + c
        zeros16 = jnp.zeros((16,), jnp.float32)

        def zero_body(i, carry):
            deg_v[pl.ds(i * 16, 16)] = zeros16
            return carry

        lax.fori_loop(0, N // 16, zero_body, 0)

        full = EPT // 16
        tail = EPT - full * 16
        dst_v[pl.ds(full * 16, 16)] = jnp.zeros((16,), jnp.int32)
        base = pl.multiple_of(wid * EPT, 8)
        pltpu.sync_copy(dst_hbm.at[pl.ds(base, EPT)], dst_v.at[pl.ds(0, EPT)])

        ones16 = jnp.ones((16,), jnp.float32)

        def count_body(i, carry):
            idx = dst_v[pl.ds(i * 16, 16)]
            plsc.addupdate_scatter(deg_v, [idx], ones16)
            return carry

        lax.fori_loop(0, full, count_body, 0)
        idx = dst_v[pl.ds(full * 16, 16)]
        mask = lax.iota(jnp.int32, 16) < tail
        plsc.addupdate_scatter(deg_v, [idx], ones16, mask=mask)

        pltpu.sync_copy(deg_v, out_hbm.at[wid])

    return deg_kernel


def _make_mp_kernel():
    mesh = plsc.VectorSubcoreMesh(core_axis_name="c", subcore_axis_name="s")

    scratch = [pltpu.VMEM_SHARED((N + 8, HH), jnp.float32)]
    scratch += [pltpu.VMEM((CH, HH), jnp.float32) for _ in range(NB)]
    scratch += [pltpu.VMEM((2, CH), jnp.int32) for _ in range(2 * NB)]
    scratch += [pltpu.SemaphoreType.DMA for _ in range(4 * NB)]

    @functools.partial(
        pl.kernel,
        out_type=jax.ShapeDtypeStruct((2 * N, HH), jnp.float32),
        mesh=mesh,
        scratch_types=scratch,
        compiler_params=pltpu.CompilerParams(use_tc_tiling_on_sc=False),
    )
    def mp_kernel(t_hbm, comb_hbm, out_hbm, acc_s, *scr):
        rows = scr[:NB]
        islot = (scr[NB:2 * NB], scr[2 * NB:3 * NB])
        gsem = scr[3 * NB:4 * NB]
        ssem = scr[4 * NB:5 * NB]
        isem = (scr[5 * NB:6 * NB], scr[6 * NB:7 * NB])
        c = lax.axis_index("c")
        s = lax.axis_index("s")
        row0 = pl.multiple_of(c * N + s * RPT, 8)
        arow0 = pl.multiple_of(s * RPT, 8)
        base = (c * NS + s) * NF

        for b in range(NB):
            pltpu.async_copy(comb_hbm.at[base + b], islot[0][b], isem[0][b])
            pltpu.async_copy(comb_hbm.at[base + NB + b], islot[1][b],
                             isem[1][b])
        pltpu.sync_copy(t_hbm.at[pl.ds(row0, RPT)], acc_s.at[pl.ds(arow0, RPT)])
        plsc.subcore_barrier()

        def wait_idx(p, b):
            pltpu.make_async_copy(comb_hbm.at[base], islot[p][b],
                                  isem[p][b]).wait()

        for b in range(NB):
            wait_idx(0, b)
            pltpu.async_copy(t_hbm.at[islot[0][b].at[0]], rows[b], gsem[b])

        def stage(j, b, p):
            pltpu.make_async_copy(t_hbm.at[islot[p][b].at[0]], rows[b],
                                  gsem[b]).wait()
            pltpu.async_copy(rows[b], acc_s.at[islot[p][b].at[1]], ssem[b],
                             add=True)

        def drain_refill(j, b, p):
            pltpu.make_async_copy(rows[b], acc_s.at[islot[p][b].at[1]],
                                  ssem[b]).wait()

            @pl.when(j + 2 * NB < NF)
            def _():
                pltpu.async_copy(comb_hbm.at[base + j + 2 * NB], islot[p][b],
                                 isem[p][b])

            @pl.when(j + NB < NF)
            def _():
                wait_idx(1 - p, b)
                pltpu.async_copy(t_hbm.at[islot[1 - p][b].at[0]], rows[b],
                                 gsem[b])

        def body(jn, carry):
            j0 = jn * 2 * NB
            ops = []
            for k in range(2 * NB):
                ops.append((stage, j0 + k, k % NB, k // NB))
            stage(*ops[0][1:])
            stage(*ops[1][1:])
            for k in range(2, 2 * NB):
                drain_refill(*ops[k - 2][1:])
                stage(*ops[k][1:])
            drain_refill(*ops[2 * NB - 2][1:])
            drain_refill(*ops[2 * NB - 1][1:])
            return carry

        lax.fori_loop(0, NF // (2 * NB), body, 0)
        plsc.subcore_barrier()
        pltpu.sync_copy(acc_s.at[pl.ds(arow0, RPT)], out_hbm.at[pl.ds(row0, RPT)])

    return mp_kernel


_make_deg_kernel = functools.cache(_make_deg_kernel)
_make_mp_kernel = functools.cache(_make_mp_kernel)


def _mm1_call(x, W, degp4):
    def body(x_ref, w_ref, degp_ref, t_ref, dinv_ref):
        deg = 1.0 + jnp.sum(degp_ref[...].reshape(NW, BM), axis=0)
        dinv = lax.rsqrt(deg)[:, None]
        hw = jnp.dot(x_ref[...], w_ref[...], preferred_element_type=jnp.float32)
        t = hw * dinv
        t_ref[0] = t[:, :HH]
        t_ref[1] = t[:, HH:]
        dinv_ref[...] = dinv

    return pl.pallas_call(
        body,
        grid=(N // BM,),
        in_specs=[
            pl.BlockSpec((BM, D), lambda i: (i, 0)),
            pl.BlockSpec((D, H), lambda i: (0, 0)),
            pl.BlockSpec((NW, 1, 1, BM), lambda i: (0, i, 0, 0)),
        ],
        out_specs=[
            pl.BlockSpec((2, BM, HH), lambda i: (0, i, 0)),
            pl.BlockSpec((BM, 1), lambda i: (i, 0)),
        ],
        out_shape=[
            jax.ShapeDtypeStruct((2, N, HH), jnp.float32),
            jax.ShapeDtypeStruct((N, 1), jnp.float32),
        ],
    )(x, W, degp4)


def _epi_mm_call(m, dinv, b, batch3, W):
    nblk = N // BM

    def body(m_ref, dinv_ref, b_ref, bat_ref, w_ref, t_ref, pool_ref, cnt_ref,
             acc, cnt):
        i = pl.program_id(0)

        @pl.when(i == 0)
        def _():
            acc[...] = jnp.zeros_like(acc)
            cnt[...] = jnp.zeros_like(cnt)

        marr = m_ref[...]
        msg = jnp.concatenate([marr[0], marr[1]], axis=1)
        dinv = dinv_ref[...]
        h = jnp.maximum(msg * dinv + b_ref[...][None, :], 0.0)

        seg = bat_ref[...].reshape(BM)
        gid = lax.broadcasted_iota(jnp.int32, (G, BM), 0)
        sel = (gid == seg[None, :]).astype(jnp.float32)
        acc[...] += jnp.dot(sel, h, preferred_element_type=jnp.float32)
        cnt[...] += jnp.sum(sel, axis=1)[:, None]

        t = jnp.dot(h, w_ref[...], preferred_element_type=jnp.float32) * dinv
        t_ref[0] = t[:, :HH]
        t_ref[1] = t[:, HH:]

        @pl.when(i == nblk - 1)
        def _():
            pool_ref[...] = acc[...]
            cnt_ref[...] = cnt[...]

    return pl.pallas_call(
        body,
        grid=(nblk,),
        in_specs=[
            pl.BlockSpec((2, BM, HH), lambda i: (0, i, 0)),
            pl.BlockSpec((BM, 1), lambda i: (i, 0)),
            pl.BlockSpec((H,), lambda i: (0,)),
            pl.BlockSpec((1, 1, BM), lambda i: (i, 0, 0)),
            pl.BlockSpec((H, H), lambda i: (0, 0)),
        ],
        out_specs=[
            pl.BlockSpec((2, BM, HH), lambda i: (0, i, 0)),
            pl.BlockSpec((G, H), lambda i: (0, 0)),
            pl.BlockSpec((G, 1), lambda i: (0, 0)),
        ],
        out_shape=[
            jax.ShapeDtypeStruct((2, N, HH), jnp.float32),
            jax.ShapeDtypeStruct((G, H), jnp.float32),
            jax.ShapeDtypeStruct((G, 1), jnp.float32),
        ],
        scratch_shapes=[
            pltpu.VMEM((G, H), jnp.float32),
            pltpu.VMEM((G, 1), jnp.float32),
        ],
    )(m, dinv, b, batch3, W)


def _epi_pool_call(m, dinv, b, batch3):
    nblk = N // BM

    def body(m_ref, dinv_ref, b_ref, bat_ref, pool_ref, acc):
        i = pl.program_id(0)

        @pl.when(i == 0)
        def _():
            acc[...] = jnp.zeros_like(acc)

        marr = m_ref[...]
        msg = jnp.concatenate([marr[0], marr[1]], axis=1)
        h = jnp.maximum(msg * dinv_ref[...] + b_ref[...][None, :], 0.0)

        seg = bat_ref[...].reshape(BM)
        gid = lax.broadcasted_iota(jnp.int32, (G, BM), 0)
        sel = (gid == seg[None, :]).astype(jnp.float32)
        acc[...] += jnp.dot(sel, h, preferred_element_type=jnp.float32)

        @pl.when(i == nblk - 1)
        def _():
            pool_ref[...] = acc[...]

    return pl.pallas_call(
        body,
        grid=(nblk,),
        in_specs=[
            pl.BlockSpec((2, BM, HH), lambda i: (0, i, 0)),
            pl.BlockSpec((BM, 1), lambda i: (i, 0)),
            pl.BlockSpec((H,), lambda i: (0,)),
            pl.BlockSpec((1, 1, BM), lambda i: (i, 0, 0)),
        ],
        out_specs=pl.BlockSpec((G, H), lambda i: (0, 0)),
        out_shape=jax.ShapeDtypeStruct((G, H), jnp.float32),
        scratch_shapes=[pltpu.VMEM((G, H), jnp.float32)],
    )(m, dinv, b, batch3)


def _head_call(p1, p2, p3, cnt, LW1, Lb1, LW2, Lb2):
    def body(p1_ref, p2_ref, p3_ref, cnt_ref, lw1_ref, lb1_ref, lw2_ref,
             lb2_ref, out_ref):
        invc = 1.0 / jnp.maximum(cnt_ref[...], 1.0)
        pool = jnp.concatenate(
            [p1_ref[...], p2_ref[...], p3_ref[...]], axis=1) * invc
        z = jnp.dot(pool, lw1_ref[...], preferred_element_type=jnp.float32)
        z = jnp.maximum(z + lb1_ref[...][None, :], 0.0)
        z2 = jnp.dot(z, lw2_ref[...], preferred_element_type=jnp.float32)
        z2 = z2 + lb2_ref[...][None, :]
        mx = jnp.max(z2, axis=1, keepdims=True)
        lse = jnp.log(jnp.sum(jnp.exp(z2 - mx), axis=1, keepdims=True)) + mx
        out_ref[...] = z2 - lse

    return pl.pallas_call(
        body,
        out_shape=jax.ShapeDtypeStruct((G, OUT), jnp.float32),
    )(p1, p2, p3, cnt, LW1, Lb1, LW2, Lb2)


def kernel(x, edge_index, batch, W1, b1, W2, b2, W3, b3, LW1, Lb1, LW2, Lb2):
    src = edge_index[0]
    dst = edge_index[1]

    degp = _make_deg_kernel()(dst)
    degp4 = degp.reshape(NW, N // BM, 1, BM)

    srcp = jnp.concatenate(
        [src.reshape(NS, EPS), jnp.zeros((NS, PADE), jnp.int32)],
        axis=1).reshape(NS, NF, CH)
    dstp = jnp.concatenate(
        [dst.reshape(NS, EPS), jnp.full((NS, PADE), N, jnp.int32)],
        axis=1).reshape(NS, NF, CH)
    src2 = jnp.stack([srcp, srcp + N], axis=0)
    dst2 = jnp.broadcast_to(dstp, (2, NS, NF, CH))
    comb = jnp.stack([src2, dst2], axis=3).reshape(2 * NS * NF, 2, CH)

    batch3 = batch.reshape(N // BM, 1, BM)
    mp = _make_mp_kernel()
    t1, dinv = _mm1_call(x, W1, degp4)
    m1 = mp(t1.reshape(2 * N, HH), comb)
    t2, p1, cnt = _epi_mm_call(m1.reshape(2, N, HH), dinv, b1, batch3, W2)
    m2 = mp(t2.reshape(2 * N, HH), comb)
    t3, p2, _ = _epi_mm_call(m2.reshape(2, N, HH), dinv, b2, batch3, W3)
    m3 = mp(t3.reshape(2 * N, HH), comb)
    p3 = _epi_pool_call(m3.reshape(2, N, HH), dinv, b3, batch3)

    return _head_call(p1, p2, p3, cnt, LW1, Lb1, LW2, Lb2)


# speedup vs baseline: 1.0799x; 1.0799x over previous
import functools

import jax
import jax.numpy as jnp
from jax import lax
from jax.experimental import pallas as pl
from jax.experimental.pallas import tpu as pltpu
from jax.experimental.pallas import tpu_sc as plsc

N = 10000
E = 160000
D = 256
H = 256
G = 128
OUT = 10

NC = 2
NS = 16
NW = NC * NS
HH = H // 2

EPT = E // NW
EPS = E // NS
CH = 40
NF = 252
PADE = NF * CH - EPS
RPT = N // NS
BM = 1000


def _make_deg_kernel():
    mesh = plsc.VectorSubcoreMesh(core_axis_name="c", subcore_axis_name="s")

    @functools.partial(
        pl.kernel,
        out_type=jax.ShapeDtypeStruct((NW, N), jnp.float32),
        mesh=mesh,
        scratch_types=[
            pltpu.VMEM((EPT + 16,), jnp.int32),
            pltpu.VMEM((N,), jnp.float32),
        ],
        compiler_params=pltpu.CompilerParams(needs_layout_passes=False,
                                             use_tc_tiling_on_sc=False),
    )
    def deg_kernel(dst_hbm, out_hbm, dst_v, deg_v):
        c = lax.axis_index("c")
        s = lax.axis_index("s")
        wid = s * NC + c
        zeros16 = jnp.zeros((16,), jnp.float32)

        def zero_body(i, carry):
            deg_v[pl.ds(i * 16, 16)] = zeros16
            return carry

        lax.fori_loop(0, N // 16, zero_body, 0)

        full = EPT // 16
        tail = EPT - full * 16
        dst_v[pl.ds(full * 16, 16)] = jnp.zeros((16,), jnp.int32)
        base = pl.multiple_of(wid * EPT, 8)
        pltpu.sync_copy(dst_hbm.at[pl.ds(base, EPT)], dst_v.at[pl.ds(0, EPT)])

        ones16 = jnp.ones((16,), jnp.float32)

        def count_body(i, carry):
            idx = dst_v[pl.ds(i * 16, 16)]
            plsc.addupdate_scatter(deg_v, [idx], ones16)
            return carry

        lax.fori_loop(0, full, count_body, 0)
        idx = dst_v[pl.ds(full * 16, 16)]
        mask = lax.iota(jnp.int32, 16) < tail
        plsc.addupdate_scatter(deg_v, [idx], ones16, mask=mask)

        pltpu.sync_copy(deg_v, out_hbm.at[wid])

    return deg_kernel


def _make_mp_kernel():
    mesh = plsc.VectorSubcoreMesh(core_axis_name="c", subcore_axis_name="s")

    @functools.partial(
        pl.kernel,
        out_type=jax.ShapeDtypeStruct((2 * N, HH), jnp.float32),
        mesh=mesh,
        scratch_types=[
            pltpu.VMEM_SHARED((N + 8, HH), jnp.float32),
            pltpu.VMEM((NF, CH), jnp.int32),
            pltpu.VMEM((NF, CH), jnp.int32),
            pltpu.VMEM((CH, HH), jnp.float32),
            pltpu.VMEM((CH, HH), jnp.float32),
            pltpu.VMEM((CH, HH), jnp.float32),
            pltpu.VMEM((CH, HH), jnp.float32),
            pltpu.VMEM((CH, HH), jnp.float32),
            pltpu.VMEM((CH, HH), jnp.float32),
            pltpu.SemaphoreType.DMA,
            pltpu.SemaphoreType.DMA,
            pltpu.SemaphoreType.DMA,
            pltpu.SemaphoreType.DMA,
            pltpu.SemaphoreType.DMA,
            pltpu.SemaphoreType.DMA,
            pltpu.SemaphoreType.DMA,
            pltpu.SemaphoreType.DMA,
            pltpu.SemaphoreType.DMA,
            pltpu.SemaphoreType.DMA,
            pltpu.SemaphoreType.DMA,
            pltpu.SemaphoreType.DMA,
        ],
        compiler_params=pltpu.CompilerParams(use_tc_tiling_on_sc=False),
    )
    def mp_kernel(t_hbm, src_hbm, dst_hbm, out_hbm, acc_s, src_v, dst_v,
                  rows0, rows1, rows2, rows3, rows4, rows5,
                  g0, g1, g2, g3, g4, g5, s0, s1, s2, s3, s4, s5):
        c = lax.axis_index("c")
        s = lax.axis_index("s")
        row0 = pl.multiple_of(c * N + s * RPT, 8)
        arow0 = pl.multiple_of(s * RPT, 8)
        pltpu.sync_copy(src_hbm.at[c * NS + s], src_v)
        pltpu.sync_copy(dst_hbm.at[s], dst_v)
        pltpu.sync_copy(t_hbm.at[pl.ds(row0, RPT)], acc_s.at[pl.ds(arow0, RPT)])
        plsc.subcore_barrier()

        bufs = ((rows0, g0, s0), (rows1, g1, s1), (rows2, g2, s2),
                (rows3, g3, s3), (rows4, g4, s4), (rows5, g5, s5))
        NB = len(bufs)
        for b, (rows_b, gb, _) in enumerate(bufs):
            pltpu.async_copy(t_hbm.at[src_v.at[b]], rows_b, gb)

        def stage(j, rows_b, gb, sb):
            pltpu.make_async_copy(t_hbm.at[src_v.at[j]], rows_b, gb).wait()
            pltpu.async_copy(rows_b, acc_s.at[dst_v.at[j]], sb, add=True)

        def drain_refill(j, rows_b, gb, sb):
            pltpu.make_async_copy(rows_b, acc_s.at[dst_v.at[j]], sb).wait()

            @pl.when(j + NB < NF)
            def _():
                pltpu.async_copy(t_hbm.at[src_v.at[j + NB]], rows_b, gb)

        def body(jn, carry):
            j = jn * NB
            stage(j, *bufs[0])
            stage(j + 1, *bufs[1])
            for b in range(2, NB):
                drain_refill(j + b - 2, *bufs[b - 2])
                stage(j + b, *bufs[b])
            drain_refill(j + NB - 2, *bufs[NB - 2])
            drain_refill(j + NB - 1, *bufs[NB - 1])
            return carry

        lax.fori_loop(0, NF // NB, body, 0)
        plsc.subcore_barrier()
        pltpu.sync_copy(acc_s.at[pl.ds(arow0, RPT)], out_hbm.at[pl.ds(row0, RPT)])

    return mp_kernel


_make_deg_kernel = functools.cache(_make_deg_kernel)
_make_mp_kernel = functools.cache(_make_mp_kernel)


def _mm1_call(x, W, degp4):
    def body(x_ref, w_ref, degp_ref, t_ref, dinv_ref):
        deg = 1.0 + jnp.sum(degp_ref[...].reshape(NW, BM), axis=0)
        dinv = lax.rsqrt(deg)[:, None]
        hw = jnp.dot(x_ref[...], w_ref[...], preferred_element_type=jnp.float32)
        t = hw * dinv
        t_ref[0] = t[:, :HH]
        t_ref[1] = t[:, HH:]
        dinv_ref[...] = dinv

    return pl.pallas_call(
        body,
        grid=(N // BM,),
        in_specs=[
            pl.BlockSpec((BM, D), lambda i: (i, 0)),
            pl.BlockSpec((D, H), lambda i: (0, 0)),
            pl.BlockSpec((NW, 1, 1, BM), lambda i: (0, i, 0, 0)),
        ],
        out_specs=[
            pl.BlockSpec((2, BM, HH), lambda i: (0, i, 0)),
            pl.BlockSpec((BM, 1), lambda i: (i, 0)),
        ],
        out_shape=[
            jax.ShapeDtypeStruct((2, N, HH), jnp.float32),
            jax.ShapeDtypeStruct((N, 1), jnp.float32),
        ],
    )(x, W, degp4)


def _epi_mm_call(m, dinv, b, batch3, W):
    nblk = N // BM

    def body(m_ref, dinv_ref, b_ref, bat_ref, w_ref, t_ref, pool_ref, cnt_ref,
             acc, cnt):
        i = pl.program_id(0)

        @pl.when(i == 0)
        def _():
            acc[...] = jnp.zeros_like(acc)
            cnt[...] = jnp.zeros_like(cnt)

        marr = m_ref[...]
        msg = jnp.concatenate([marr[0], marr[1]], axis=1)
        dinv = dinv_ref[...]
        h = jnp.maximum(msg * dinv + b_ref[...][None, :], 0.0)

        seg = bat_ref[...].reshape(BM)
        gid = lax.broadcasted_iota(jnp.int32, (G, BM), 0)
        sel = (gid == seg[None, :]).astype(jnp.float32)
        acc[...] += jnp.dot(sel, h, preferred_element_type=jnp.float32)
        cnt[...] += jnp.sum(sel, axis=1)[:, None]

        t = jnp.dot(h, w_ref[...], preferred_element_type=jnp.float32) * dinv
        t_ref[0] = t[:, :HH]
        t_ref[1] = t[:, HH:]

        @pl.when(i == nblk - 1)
        def _():
            pool_ref[...] = acc[...]
            cnt_ref[...] = cnt[...]

    return pl.pallas_call(
        body,
        grid=(nblk,),
        in_specs=[
            pl.BlockSpec((2, BM, HH), lambda i: (0, i, 0)),
            pl.BlockSpec((BM, 1), lambda i: (i, 0)),
            pl.BlockSpec((H,), lambda i: (0,)),
            pl.BlockSpec((1, 1, BM), lambda i: (i, 0, 0)),
            pl.BlockSpec((H, H), lambda i: (0, 0)),
        ],
        out_specs=[
            pl.BlockSpec((2, BM, HH), lambda i: (0, i, 0)),
            pl.BlockSpec((G, H), lambda i: (0, 0)),
            pl.BlockSpec((G, 1), lambda i: (0, 0)),
        ],
        out_shape=[
            jax.ShapeDtypeStruct((2, N, HH), jnp.float32),
            jax.ShapeDtypeStruct((G, H), jnp.float32),
            jax.ShapeDtypeStruct((G, 1), jnp.float32),
        ],
        scratch_shapes=[
            pltpu.VMEM((G, H), jnp.float32),
            pltpu.VMEM((G, 1), jnp.float32),
        ],
    )(m, dinv, b, batch3, W)


def _epi_pool_call(m, dinv, b, batch3):
    nblk = N // BM

    def body(m_ref, dinv_ref, b_ref, bat_ref, pool_ref, acc):
        i = pl.program_id(0)

        @pl.when(i == 0)
        def _():
            acc[...] = jnp.zeros_like(acc)

        marr = m_ref[...]
        msg = jnp.concatenate([marr[0], marr[1]], axis=1)
        h = jnp.maximum(msg * dinv_ref[...] + b_ref[...][None, :], 0.0)

        seg = bat_ref[...].reshape(BM)
        gid = lax.broadcasted_iota(jnp.int32, (G, BM), 0)
        sel = (gid == seg[None, :]).astype(jnp.float32)
        acc[...] += jnp.dot(sel, h, preferred_element_type=jnp.float32)

        @pl.when(i == nblk - 1)
        def _():
            pool_ref[...] = acc[...]

    return pl.pallas_call(
        body,
        grid=(nblk,),
        in_specs=[
            pl.BlockSpec((2, BM, HH), lambda i: (0, i, 0)),
            pl.BlockSpec((BM, 1), lambda i: (i, 0)),
            pl.BlockSpec((H,), lambda i: (0,)),
            pl.BlockSpec((1, 1, BM), lambda i: (i, 0, 0)),
        ],
        out_specs=pl.BlockSpec((G, H), lambda i: (0, 0)),
        out_shape=jax.ShapeDtypeStruct((G, H), jnp.float32),
        scratch_shapes=[pltpu.VMEM((G, H), jnp.float32)],
    )(m, dinv, b, batch3)


def _head_call(p1, p2, p3, cnt, LW1, Lb1, LW2, Lb2):
    def body(p1_ref, p2_ref, p3_ref, cnt_ref, lw1_ref, lb1_ref, lw2_ref,
             lb2_ref, out_ref):
        invc = 1.0 / jnp.maximum(cnt_ref[...], 1.0)
        pool = jnp.concatenate(
            [p1_ref[...], p2_ref[...], p3_ref[...]], axis=1) * invc
        z = jnp.dot(pool, lw1_ref[...], preferred_element_type=jnp.float32)
        z = jnp.maximum(z + lb1_ref[...][None, :], 0.0)
        z2 = jnp.dot(z, lw2_ref[...], preferred_element_type=jnp.float32)
        z2 = z2 + lb2_ref[...][None, :]
        mx = jnp.max(z2, axis=1, keepdims=True)
        lse = jnp.log(jnp.sum(jnp.exp(z2 - mx), axis=1, keepdims=True)) + mx
        out_ref[...] = z2 - lse

    return pl.pallas_call(
        body,
        out_shape=jax.ShapeDtypeStruct((G, OUT), jnp.float32),
    )(p1, p2, p3, cnt, LW1, Lb1, LW2, Lb2)


def kernel(x, edge_index, batch, W1, b1, W2, b2, W3, b3, LW1, Lb1, LW2, Lb2):
    src = edge_index[0]
    dst = edge_index[1]

    degp = _make_deg_kernel()(dst)
    degp4 = degp.reshape(NW, N // BM, 1, BM)

    srcp = jnp.concatenate(
        [src.reshape(NS, EPS), jnp.zeros((NS, PADE), jnp.int32)], axis=1)
    src4 = jnp.concatenate([srcp, srcp + N], axis=0).reshape(2 * NS, NF, CH)
    dst3 = jnp.concatenate(
        [dst.reshape(NS, EPS), jnp.full((NS, PADE), N, jnp.int32)],
        axis=1).reshape(NS, NF, CH)

    batch3 = batch.reshape(N // BM, 1, BM)
    mp = _make_mp_kernel()
    t1, dinv = _mm1_call(x, W1, degp4)
    m1 = mp(t1.reshape(2 * N, HH), src4, dst3)
    t2, p1, cnt = _epi_mm_call(m1.reshape(2, N, HH), dinv, b1, batch3, W2)
    m2 = mp(t2.reshape(2 * N, HH), src4, dst3)
    t3, p2, _ = _epi_mm_call(m2.reshape(2, N, HH), dinv, b2, batch3, W3)
    m3 = mp(t3.reshape(2 * N, HH), src4, dst3)
    p3 = _epi_pool_call(m3.reshape(2, N, HH), dinv, b3, batch3)

    return _head_call(p1, p2, p3, cnt, LW1, Lb1, LW2, Lb2)


# speedup vs baseline: 1.0880x; 1.0076x over previous
import functools

import jax
import jax.numpy as jnp
from jax import lax
from jax.experimental import pallas as pl
from jax.experimental.pallas import tpu as pltpu
from jax.experimental.pallas import tpu_sc as plsc

N = 10000
E = 160000
D = 256
H = 256
G = 128
OUT = 10

NC = 2
NS = 16
NW = NC * NS
HH = H // 2

EPT = E // NW
EPS = E // NS
CH = 32
NF = 315
PADE = NF * CH - EPS
RPT = N // NS
BM = 1000


def _make_deg_kernel():
    mesh = plsc.VectorSubcoreMesh(core_axis_name="c", subcore_axis_name="s")

    @functools.partial(
        pl.kernel,
        out_type=jax.ShapeDtypeStruct((NW, N), jnp.float32),
        mesh=mesh,
        scratch_types=[
            pltpu.VMEM((EPT + 16,), jnp.int32),
            pltpu.VMEM((N,), jnp.float32),
        ],
        compiler_params=pltpu.CompilerParams(needs_layout_passes=False,
                                             use_tc_tiling_on_sc=False),
    )
    def deg_kernel(dst_hbm, out_hbm, dst_v, deg_v):
        c = lax.axis_index("c")
        s = lax.axis_index("s")
        wid = s * NC + c
        zeros16 = jnp.zeros((16,), jnp.float32)

        def zero_body(i, carry):
            deg_v[pl.ds(i * 16, 16)] = zeros16
            return carry

        lax.fori_loop(0, N // 16, zero_body, 0)

        full = EPT // 16
        tail = EPT - full * 16
        dst_v[pl.ds(full * 16, 16)] = jnp.zeros((16,), jnp.int32)
        base = pl.multiple_of(wid * EPT, 8)
        pltpu.sync_copy(dst_hbm.at[pl.ds(base, EPT)], dst_v.at[pl.ds(0, EPT)])

        ones16 = jnp.ones((16,), jnp.float32)

        def count_body(i, carry):
            idx = dst_v[pl.ds(i * 16, 16)]
            plsc.addupdate_scatter(deg_v, [idx], ones16)
            return carry

        lax.fori_loop(0, full, count_body, 0)
        idx = dst_v[pl.ds(full * 16, 16)]
        mask = lax.iota(jnp.int32, 16) < tail
        plsc.addupdate_scatter(deg_v, [idx], ones16, mask=mask)

        pltpu.sync_copy(deg_v, out_hbm.at[wid])

    return deg_kernel


def _make_mp_kernel():
    mesh = plsc.VectorSubcoreMesh(core_axis_name="c", subcore_axis_name="s")

    @functools.partial(
        pl.kernel,
        out_type=jax.ShapeDtypeStruct((2 * N, HH), jnp.float32),
        mesh=mesh,
        scratch_types=[
            pltpu.VMEM_SHARED((N + 8, HH), jnp.float32),
            pltpu.VMEM((NF, CH), jnp.int32),
            pltpu.VMEM((NF, CH), jnp.int32),
            pltpu.VMEM((CH, HH), jnp.float32),
            pltpu.VMEM((CH, HH), jnp.float32),
            pltpu.VMEM((CH, HH), jnp.float32),
            pltpu.VMEM((CH, HH), jnp.float32),
            pltpu.VMEM((CH, HH), jnp.float32),
            pltpu.VMEM((CH, HH), jnp.float32),
            pltpu.VMEM((CH, HH), jnp.float32),
            pltpu.SemaphoreType.DMA,
            pltpu.SemaphoreType.DMA,
            pltpu.SemaphoreType.DMA,
            pltpu.SemaphoreType.DMA,
            pltpu.SemaphoreType.DMA,
            pltpu.SemaphoreType.DMA,
            pltpu.SemaphoreType.DMA,
            pltpu.SemaphoreType.DMA,
            pltpu.SemaphoreType.DMA,
            pltpu.SemaphoreType.DMA,
            pltpu.SemaphoreType.DMA,
            pltpu.SemaphoreType.DMA,
            pltpu.SemaphoreType.DMA,
            pltpu.SemaphoreType.DMA,
        ],
        compiler_params=pltpu.CompilerParams(use_tc_tiling_on_sc=False),
    )
    def mp_kernel(t_hbm, src_hbm, dst_hbm, out_hbm, acc_s, src_v, dst_v,
                  rows0, rows1, rows2, rows3, rows4, rows5, rows6,
                  g0, g1, g2, g3, g4, g5, g6, s0, s1, s2, s3, s4, s5, s6):
        c = lax.axis_index("c")
        s = lax.axis_index("s")
        row0 = pl.multiple_of(c * N + s * RPT, 8)
        arow0 = pl.multiple_of(s * RPT, 8)
        pltpu.sync_copy(src_hbm.at[c * NS + s], src_v)
        pltpu.sync_copy(dst_hbm.at[s], dst_v)
        pltpu.sync_copy(t_hbm.at[pl.ds(row0, RPT)], acc_s.at[pl.ds(arow0, RPT)])
        plsc.subcore_barrier()

        bufs = ((rows0, g0, s0), (rows1, g1, s1), (rows2, g2, s2),
                (rows3, g3, s3), (rows4, g4, s4), (rows5, g5, s5),
                (rows6, g6, s6))
        NB = len(bufs)
        for b, (rows_b, gb, _) in enumerate(bufs):
            pltpu.async_copy(t_hbm.at[src_v.at[b]], rows_b, gb)

        def stage(j, rows_b, gb, sb):
            pltpu.make_async_copy(t_hbm.at[src_v.at[j]], rows_b, gb).wait()
            pltpu.async_copy(rows_b, acc_s.at[dst_v.at[j]], sb, add=True)

        def drain_refill(j, rows_b, gb, sb):
            pltpu.make_async_copy(rows_b, acc_s.at[dst_v.at[j]], sb).wait()

            @pl.when(j + NB < NF)
            def _():
                pltpu.async_copy(t_hbm.at[src_v.at[j + NB]], rows_b, gb)

        def body(jn, carry):
            j = jn * NB
            stage(j, *bufs[0])
            stage(j + 1, *bufs[1])
            for b in range(2, NB):
                drain_refill(j + b - 2, *bufs[b - 2])
                stage(j + b, *bufs[b])
            drain_refill(j + NB - 2, *bufs[NB - 2])
            drain_refill(j + NB - 1, *bufs[NB - 1])
            return carry

        lax.fori_loop(0, NF // NB, body, 0)
        plsc.subcore_barrier()
        pltpu.sync_copy(acc_s.at[pl.ds(arow0, RPT)], out_hbm.at[pl.ds(row0, RPT)])

    return mp_kernel


_make_deg_kernel = functools.cache(_make_deg_kernel)
_make_mp_kernel = functools.cache(_make_mp_kernel)


def _mm1_call(x, W, degp4):
    def body(x_ref, w_ref, degp_ref, t_ref, dinv_ref):
        deg = 1.0 + jnp.sum(degp_ref[...].reshape(NW, BM), axis=0)
        dinv = lax.rsqrt(deg)[:, None]
        hw = jnp.dot(x_ref[...], w_ref[...], preferred_element_type=jnp.float32)
        t = hw * dinv
        t_ref[0] = t[:, :HH]
        t_ref[1] = t[:, HH:]
        dinv_ref[...] = dinv

    return pl.pallas_call(
        body,
        grid=(N // BM,),
        in_specs=[
            pl.BlockSpec((BM, D), lambda i: (i, 0)),
            pl.BlockSpec((D, H), lambda i: (0, 0)),
            pl.BlockSpec((NW, 1, 1, BM), lambda i: (0, i, 0, 0)),
        ],
        out_specs=[
            pl.BlockSpec((2, BM, HH), lambda i: (0, i, 0)),
            pl.BlockSpec((BM, 1), lambda i: (i, 0)),
        ],
        out_shape=[
            jax.ShapeDtypeStruct((2, N, HH), jnp.float32),
            jax.ShapeDtypeStruct((N, 1), jnp.float32),
        ],
    )(x, W, degp4)


def _epi_mm_call(m, dinv, b, batch3, W):
    nblk = N // BM

    def body(m_ref, dinv_ref, b_ref, bat_ref, w_ref, t_ref, pool_ref, cnt_ref,
             acc, cnt):
        i = pl.program_id(0)

        @pl.when(i == 0)
        def _():
            acc[...] = jnp.zeros_like(acc)
            cnt[...] = jnp.zeros_like(cnt)

        marr = m_ref[...]
        msg = jnp.concatenate([marr[0], marr[1]], axis=1)
        dinv = dinv_ref[...]
        h = jnp.maximum(msg * dinv + b_ref[...][None, :], 0.0)

        seg = bat_ref[...].reshape(BM)
        gid = lax.broadcasted_iota(jnp.int32, (G, BM), 0)
        sel = (gid == seg[None, :]).astype(jnp.float32)
        acc[...] += jnp.dot(sel, h, preferred_element_type=jnp.float32)
        cnt[...] += jnp.sum(sel, axis=1)[:, None]

        t = jnp.dot(h, w_ref[...], preferred_element_type=jnp.float32) * dinv
        t_ref[0] = t[:, :HH]
        t_ref[1] = t[:, HH:]

        @pl.when(i == nblk - 1)
        def _():
            pool_ref[...] = acc[...]
            cnt_ref[...] = cnt[...]

    return pl.pallas_call(
        body,
        grid=(nblk,),
        in_specs=[
            pl.BlockSpec((2, BM, HH), lambda i: (0, i, 0)),
            pl.BlockSpec((BM, 1), lambda i: (i, 0)),
            pl.BlockSpec((H,), lambda i: (0,)),
            pl.BlockSpec((1, 1, BM), lambda i: (i, 0, 0)),
            pl.BlockSpec((H, H), lambda i: (0, 0)),
        ],
        out_specs=[
            pl.BlockSpec((2, BM, HH), lambda i: (0, i, 0)),
            pl.BlockSpec((G, H), lambda i: (0, 0)),
            pl.BlockSpec((G, 1), lambda i: (0, 0)),
        ],
        out_shape=[
            jax.ShapeDtypeStruct((2, N, HH), jnp.float32),
            jax.ShapeDtypeStruct((G, H), jnp.float32),
            jax.ShapeDtypeStruct((G, 1), jnp.float32),
        ],
        scratch_shapes=[
            pltpu.VMEM((G, H), jnp.float32),
            pltpu.VMEM((G, 1), jnp.float32),
        ],
    )(m, dinv, b, batch3, W)


def _epi_pool_call(m, dinv, b, batch3):
    nblk = N // BM

    def body(m_ref, dinv_ref, b_ref, bat_ref, pool_ref, acc):
        i = pl.program_id(0)

        @pl.when(i == 0)
        def _():
            acc[...] = jnp.zeros_like(acc)

        marr = m_ref[...]
        msg = jnp.concatenate([marr[0], marr[1]], axis=1)
        h = jnp.maximum(msg * dinv_ref[...] + b_ref[...][None, :], 0.0)

        seg = bat_ref[...].reshape(BM)
        gid = lax.broadcasted_iota(jnp.int32, (G, BM), 0)
        sel = (gid == seg[None, :]).astype(jnp.float32)
        acc[...] += jnp.dot(sel, h, preferred_element_type=jnp.float32)

        @pl.when(i == nblk - 1)
        def _():
            pool_ref[...] = acc[...]

    return pl.pallas_call(
        body,
        grid=(nblk,),
        in_specs=[
            pl.BlockSpec((2, BM, HH), lambda i: (0, i, 0)),
            pl.BlockSpec((BM, 1), lambda i: (i, 0)),
            pl.BlockSpec((H,), lambda i: (0,)),
            pl.BlockSpec((1, 1, BM), lambda i: (i, 0, 0)),
        ],
        out_specs=pl.BlockSpec((G, H), lambda i: (0, 0)),
        out_shape=jax.ShapeDtypeStruct((G, H), jnp.float32),
        scratch_shapes=[pltpu.VMEM((G, H), jnp.float32)],
    )(m, dinv, b, batch3)


def _head_call(p1, p2, p3, cnt, LW1, Lb1, LW2, Lb2):
    def body(p1_ref, p2_ref, p3_ref, cnt_ref, lw1_ref, lb1_ref, lw2_ref,
             lb2_ref, out_ref):
        invc = 1.0 / jnp.maximum(cnt_ref[...], 1.0)
        pool = jnp.concatenate(
            [p1_ref[...], p2_ref[...], p3_ref[...]], axis=1) * invc
        z = jnp.dot(pool, lw1_ref[...], preferred_element_type=jnp.float32)
        z = jnp.maximum(z + lb1_ref[...][None, :], 0.0)
        z2 = jnp.dot(z, lw2_ref[...], preferred_element_type=jnp.float32)
        z2 = z2 + lb2_ref[...][None, :]
        mx = jnp.max(z2, axis=1, keepdims=True)
        lse = jnp.log(jnp.sum(jnp.exp(z2 - mx), axis=1, keepdims=True)) + mx
        out_ref[...] = z2 - lse

    return pl.pallas_call(
        body,
        out_shape=jax.ShapeDtypeStruct((G, OUT), jnp.float32),
    )(p1, p2, p3, cnt, LW1, Lb1, LW2, Lb2)


def kernel(x, edge_index, batch, W1, b1, W2, b2, W3, b3, LW1, Lb1, LW2, Lb2):
    src = edge_index[0]
    dst = edge_index[1]

    degp = _make_deg_kernel()(dst)
    degp4 = degp.reshape(NW, N // BM, 1, BM)

    srcp = jnp.concatenate(
        [src.reshape(NS, EPS), jnp.zeros((NS, PADE), jnp.int32)], axis=1)
    src4 = jnp.concatenate([srcp, srcp + N], axis=0).reshape(2 * NS, NF, CH)
    dst3 = jnp.concatenate(
        [dst.reshape(NS, EPS), jnp.full((NS, PADE), N, jnp.int32)],
        axis=1).reshape(NS, NF, CH)

    batch3 = batch.reshape(N // BM, 1, BM)
    mp = _make_mp_kernel()
    t1, dinv = _mm1_call(x, W1, degp4)
    m1 = mp(t1.reshape(2 * N, HH), src4, dst3)
    t2, p1, cnt = _epi_mm_call(m1.reshape(2, N, HH), dinv, b1, batch3, W2)
    m2 = mp(t2.reshape(2 * N, HH), src4, dst3)
    t3, p2, _ = _epi_mm_call(m2.reshape(2, N, HH), dinv, b2, batch3, W3)
    m3 = mp(t3.reshape(2 * N, HH), src4, dst3)
    p3 = _epi_pool_call(m3.reshape(2, N, HH), dinv, b3, batch3)

    return _head_call(p1, p2, p3, cnt, LW1, Lb1, LW2, Lb2)
